# baseline (speedup 1.0000x reference)
import functools

import jax
import jax.numpy as jnp
from jax import lax
from jax.experimental import pallas as pl
from jax.experimental.pallas import tpu as pltpu
from jax.experimental.pallas import tpu_sc as plsc

NC, NS, L = 2, 16, 16
NW = NC * NS
B, T, D = 16384, 50, 32
BAGS_PER_STEP = 2
ROWS_PER_STEP = BAGS_PER_STEP * T
BAGS_PER_W = B // NW
STEPS = BAGS_PER_W // BAGS_PER_STEP
STEP_STRIDE = 104
NBUF = 8


def _embedding_bag_sc(idx_flat, emb_table):
    mesh = plsc.VectorSubcoreMesh(core_axis_name="c", subcore_axis_name="s")

    IDX_PER_W = STEPS * STEP_STRIDE

    @functools.partial(
        pl.kernel,
        mesh=mesh,
        out_type=jax.ShapeDtypeStruct((B * D,), jnp.float32),
        compiler_params=pltpu.CompilerParams(use_tc_tiling_on_sc=False),
        scratch_types=[
            pltpu.VMEM((IDX_PER_W,), jnp.int32),
            pltpu.VMEM((NBUF, ROWS_PER_STEP, D), jnp.float32),
            pltpu.VMEM((BAGS_PER_W * D,), jnp.float32),
            pltpu.SemaphoreType.DMA((NBUF,)),
        ],
    )
    def bag_kernel(idx_hbm, table_hbm, out_hbm, idx_v, rows_v, out_v, sem):
        wid = lax.axis_index("s") * NC + lax.axis_index("c")
        pltpu.sync_copy(idx_hbm.at[pl.ds(wid * IDX_PER_W, IDX_PER_W)], idx_v)

        for b in range(NBUF):
            pltpu.make_async_copy(
                table_hbm.at[idx_v.at[pl.ds(b * STEP_STRIDE, ROWS_PER_STEP)]],
                rows_v.at[b], sem.at[b]).start()

        @pl.loop(0, STEPS, step=NBUF)
        def _(j0):
            for b in range(NBUF):
                j = j0 + b
                buf = rows_v.at[b]
                pltpu.make_async_copy(
                    table_hbm.at[idx_v.at[pl.ds(j * STEP_STRIDE, ROWS_PER_STEP)]],
                    buf, sem.at[b]).wait()
                for bag in range(BAGS_PER_STEP):
                    for h in range(D // L):
                        acc0 = buf[bag * T, pl.ds(h * L, L)]
                        acc1 = buf[bag * T + 1, pl.ds(h * L, L)]
                        for r in range(2, T, 2):
                            acc0 = acc0 + buf[bag * T + r, pl.ds(h * L, L)]
                            acc1 = acc1 + buf[bag * T + r + 1, pl.ds(h * L, L)]
                        off = (j * BAGS_PER_STEP + bag) * D + h * L
                        out_v[pl.ds(off, L)] = acc0 + acc1

                @pl.when(j + NBUF < STEPS)
                def _():
                    pltpu.make_async_copy(
                        table_hbm.at[idx_v.at[pl.ds((j + NBUF) * STEP_STRIDE,
                                                    ROWS_PER_STEP)]],
                        buf, sem.at[b]).start()

        pltpu.sync_copy(out_v, out_hbm.at[pl.ds(wid * BAGS_PER_W * D, BAGS_PER_W * D)])

    return bag_kernel(idx_flat, emb_table)


PACK = 4
BKP = 512
H1, H2 = 128, 64


def _mlp_body(x_ref, e_ref, w1a, w1b, b1r, g1r, be1r, w2, b2r, g2r, be2r, w3, b3r, o_ref):
    h = jnp.dot(x_ref[...], w1a[...], preferred_element_type=jnp.float32)
    h = h + jnp.dot(e_ref[...], w1b[...], preferred_element_type=jnp.float32)
    h = h + b1r[...]
    h = jnp.maximum(h, 0.0)
    parts = []
    for s in range(PACK):
        hs = h[:, s * H1:(s + 1) * H1]
        mu = jnp.mean(hs, axis=-1, keepdims=True)
        var = jnp.mean((hs - mu) ** 2, axis=-1, keepdims=True)
        parts.append((hs - mu) / jnp.sqrt(var + 1e-5) * g1r[...] + be1r[...])
    h = jnp.concatenate(parts, axis=1)
    h = jnp.dot(h, w2[...], preferred_element_type=jnp.float32) + b2r[...]
    h = jnp.maximum(h, 0.0)
    parts = []
    for s in range(PACK):
        hs = h[:, s * H2:(s + 1) * H2]
        mu = jnp.mean(hs, axis=-1, keepdims=True)
        var = jnp.mean((hs - mu) ** 2, axis=-1, keepdims=True)
        parts.append((hs - mu) / jnp.sqrt(var + 1e-5) * g2r[...] + be2r[...])
    h = jnp.concatenate(parts, axis=1)
    o_ref[...] = jnp.dot(h, w3[...], preferred_element_type=jnp.float32) + b3r[...]


def _mlp_tc(xp, e2d, W1A, W1B, b1q, g1r, be1r, W2bd, b2q, g2r, be2r, W3bd, b3q):
    np_rows = B // PACK
    full = lambda a: pl.BlockSpec(a.shape, lambda i: (0, 0))
    return pl.pallas_call(
        _mlp_body,
        grid=(np_rows // BKP,),
        in_specs=[
            pl.BlockSpec((BKP, xp.shape[1]), lambda i: (i, 0)),
            pl.BlockSpec((BKP, D * PACK), lambda i: (i, 0)),
            full(W1A), full(W1B), full(b1q), full(g1r), full(be1r),
            full(W2bd), full(b2q), full(g2r), full(be2r),
            full(W3bd), full(b3q),
        ],
        out_specs=pl.BlockSpec((BKP, 8 * PACK), lambda i: (i, 0)),
        out_shape=jax.ShapeDtypeStruct((np_rows, 8 * PACK), jnp.float32),
    )(xp, e2d, W1A, W1B, b1q, g1r, be1r, W2bd, b2q, g2r, be2r, W3bd, b3q)


def _blockdiag(w):
    a, b = w.shape
    out = jnp.zeros((PACK * a, PACK * b), jnp.float32)
    for s in range(PACK):
        out = out.at[s * a:(s + 1) * a, s * b:(s + 1) * b].set(w)
    return out


def kernel(x_num, leaf_ids, emb_table, W1, b1, g1, be1, W2, b2, g2, be2, W3, b3):
    idx2d = leaf_ids.astype(jnp.int32).reshape(B * T // ROWS_PER_STEP, ROWS_PER_STEP)
    idx_flat = jnp.zeros((idx2d.shape[0], STEP_STRIDE), jnp.int32)
    idx_flat = idx_flat.at[:, :ROWS_PER_STEP].set(idx2d).reshape(-1)
    emb_flat = _embedding_bag_sc(idx_flat, emb_table)
    e2d = emb_flat.reshape(B * D // (D * PACK), D * PACK)

    n_feat = x_num.shape[1]
    W1a, W1b = W1[:n_feat], W1[n_feat:]
    W3p = jnp.zeros((W3.shape[0], 8), jnp.float32).at[:, :3].set(W3)
    b3p = jnp.zeros((8,), jnp.float32).at[:3].set(b3)

    xp = x_num.reshape(B // PACK, PACK * n_feat)
    out = _mlp_tc(
        xp, e2d,
        _blockdiag(W1a), _blockdiag(W1b),
        jnp.tile(b1, PACK).reshape(1, -1), g1.reshape(1, -1), be1.reshape(1, -1),
        _blockdiag(W2), jnp.tile(b2, PACK).reshape(1, -1),
        g2.reshape(1, -1), be2.reshape(1, -1),
        _blockdiag(W3p), jnp.tile(b3p, PACK).reshape(1, -1),
    )
    return out.reshape(B, 8)[:, :3]
